# baseline (speedup 1.0000x reference)
import jax
import jax.numpy as jnp
from jax import lax
from jax.experimental import pallas as pl

B, D, C = 16384, 1024, 51


def _head(x_ref, w_ref, b_ref, bins_ref, pt_ref, val_ref, col0, blk):
    lt = lax.dot_general(
        w_ref[...], x_ref[...],
        (((1,), (1,)), ((), ())),
        preferred_element_type=jnp.float32,
    )
    lt = lt + b_ref[...]
    m = jnp.max(lt, axis=0, keepdims=True)
    e = jnp.exp(lt - m)
    s = jnp.sum(e, axis=0, keepdims=True)
    rinv = 1.0 / s
    num = jnp.sum(e * bins_ref[...], axis=0, keepdims=True)
    pt_ref[:, col0:col0 + blk] = e * rinv
    val_ref[0, 0, col0:col0 + blk] = (num * rinv)[0, :]


def _head_kernel(x0_ref, x1_ref, w_ref, b_ref, bins_ref, pt_ref, val_ref):
    blk = x0_ref.shape[0]
    _head(x0_ref, w_ref, b_ref, bins_ref, pt_ref, val_ref, 0, blk)
    _head(x1_ref, w_ref, b_ref, bins_ref, pt_ref, val_ref, blk, blk)


@jax.jit
def kernel(x, W, b, bins):
    blk = 512
    nb = B // blk
    b2 = b.reshape(C, 1)
    bins2 = bins.reshape(C, 1)
    pt, val = pl.pallas_call(
        _head_kernel,
        grid=(nb // 2,),
        in_specs=[
            pl.BlockSpec((blk, D), lambda i: (2 * i, 0)),
            pl.BlockSpec((blk, D), lambda i: (2 * i + 1, 0)),
            pl.BlockSpec((C, D), lambda i: (0, 0)),
            pl.BlockSpec((C, 1), lambda i: (0, 0)),
            pl.BlockSpec((C, 1), lambda i: (0, 0)),
        ],
        out_specs=[
            pl.BlockSpec((C, 2 * blk), lambda i: (0, i)),
            pl.BlockSpec((1, 1, 2 * blk), lambda i: (i, 0, 0)),
        ],
        out_shape=[
            jax.ShapeDtypeStruct((C, B), jnp.float32),
            jax.ShapeDtypeStruct((nb // 2, 1, 2 * blk), jnp.float32),
        ],
    )(x, x, W, b2, bins2)
    return pt.T, val.reshape(B)


# speedup vs baseline: 1.0666x; 1.0666x over previous
import jax
import jax.numpy as jnp
from jax import lax
from jax.experimental import pallas as pl

B, D, C = 16384, 1024, 51
NSTREAM = 4
BLK = 1024


def _head(x_ref, w_ref, b_ref, bins_ref, pt_ref, val_ref, col0):
    lt = lax.dot_general(
        w_ref[...], x_ref[...],
        (((1,), (1,)), ((), ())),
        preferred_element_type=jnp.float32,
    )
    lt = lt + b_ref[...]
    m = jnp.max(lt, axis=0, keepdims=True)
    e = jnp.exp(lt - m)
    s = jnp.sum(e, axis=0, keepdims=True)
    rinv = 1.0 / s
    num = jnp.sum(e * bins_ref[...], axis=0, keepdims=True)
    pt_ref[:, col0:col0 + BLK] = e * rinv
    val_ref[0, 0, col0:col0 + BLK] = (num * rinv)[0, :]


def _head_kernel(*refs):
    x_refs = refs[:NSTREAM]
    w_ref, b_ref, bins_ref, pt_ref, val_ref = refs[NSTREAM:]
    for j in range(NSTREAM):
        _head(x_refs[j], w_ref, b_ref, bins_ref, pt_ref, val_ref, j * BLK)


def _mk_in_spec(j):
    return pl.BlockSpec((BLK, D), lambda i, j=j: (NSTREAM * i + j, 0))


@jax.jit
def kernel(x, W, b, bins):
    cols = NSTREAM * BLK
    ng = B // cols
    b2 = b.reshape(C, 1)
    bins2 = bins.reshape(C, 1)
    pt, val = pl.pallas_call(
        _head_kernel,
        grid=(ng,),
        in_specs=[_mk_in_spec(j) for j in range(NSTREAM)] + [
            pl.BlockSpec((C, D), lambda i: (0, 0)),
            pl.BlockSpec((C, 1), lambda i: (0, 0)),
            pl.BlockSpec((C, 1), lambda i: (0, 0)),
        ],
        out_specs=[
            pl.BlockSpec((C, cols), lambda i: (0, i)),
            pl.BlockSpec((1, 1, cols), lambda i: (i, 0, 0)),
        ],
        out_shape=[
            jax.ShapeDtypeStruct((C, B), jnp.float32),
            jax.ShapeDtypeStruct((ng, 1, cols), jnp.float32),
        ],
    )(*([x] * NSTREAM), W, b2, bins2)
    return pt.T, val.reshape(B)
